# baseline (speedup 1.0000x reference)
import functools

import jax
import jax.numpy as jnp
from jax import lax
from jax.experimental import pallas as pl
from jax.experimental.pallas import tpu as pltpu
from jax.experimental.pallas import tpu_sc as plsc

_L = 16
_NC = 2
_NS = 16
_NW = _NC * _NS

_CHUNK = 256
_IDX_ROWS = _CHUNK // 128
_GRP = 8
_NBUF = 2


def _sc_lookup(day2d, table, *, b_per_w):
    n_chunks = b_per_w // _CHUNK
    n_grps = n_chunks // _GRP
    n_iters = n_grps // 2
    total = day2d.shape[0] * 128
    grp_rows = _GRP * _IDX_ROWS

    mesh = plsc.VectorSubcoreMesh(core_axis_name="c", subcore_axis_name="s")

    @functools.partial(
        pl.kernel,
        mesh=mesh,
        out_type=jax.ShapeDtypeStruct((total, 128), jnp.float32),
        scratch_types=[
            pltpu.VMEM_SHARED((8, 128), jnp.float32),
            pltpu.VMEM((2, grp_rows, 128), jnp.int32),
            pltpu.VMEM((_NBUF, _CHUNK, 128), jnp.float32),
            pltpu.SemaphoreType.DMA,
            pltpu.SemaphoreType.DMA,
            pltpu.SemaphoreType.DMA,
            pltpu.SemaphoreType.DMA,
            pltpu.SemaphoreType.DMA,
        ],
    )
    def k(day_hbm, table_hbm, out_hbm, tbl_s, idx_v, rows_v, sem_d0,
          sem_d1, sem_g, sem_o0, sem_o1):
        sid = lax.axis_index("s")
        wid = sid * _NC + lax.axis_index("c")
        chunk_base0 = wid * n_chunks

        @pl.when(sid == 0)
        def _():
            pltpu.sync_copy(table_hbm, tbl_s)

        plsc.subcore_barrier()

        day_sems = (sem_d0, sem_d1)
        out_sems = (sem_o0, sem_o1)

        def day_copy(grp, p):
            row0 = (chunk_base0 + grp * _GRP) * _IDX_ROWS
            return pltpu.make_async_copy(
                day_hbm.at[pl.ds(row0, grp_rows)], idx_v.at[p], day_sems[p])

        def gather_copy(p, k_, j):
            return pltpu.make_async_copy(
                tbl_s.at[idx_v.at[p, k_ * _IDX_ROWS + j]],
                rows_v.at[k_ % _NBUF, pl.ds(j * 128, 128)],
                sem_g,
            )

        def out_half(k_, chunk, j):
            return pltpu.make_async_copy(
                rows_v.at[k_ % _NBUF, pl.ds(j * 128, 128)],
                out_hbm.at[pl.ds(chunk * _CHUNK + j * 128, 128)],
                out_sems[k_ % _NBUF],
            )

        def remap(p, k_):
            for j in range(_IDX_ROWS):
                row = k_ * _IDX_ROWS + j
                for kk in range(128 // _L):
                    d = idx_v[p, row, pl.ds(kk * _L, _L)]
                    q = ((d.astype(jnp.float32) + 0.5) * (1.0 / 7.0)
                         ).astype(jnp.int32)
                    r = d - q * 7 + 1
                    r = jnp.where(d == 0, 0, r)
                    idx_v[p, row, pl.ds(kk * _L, _L)] = r

        day_copy(0, 0).start()
        day_copy(1, 1).start()

        def body(i, _):
            for p in range(2):
                g = i * 2 + p
                day_copy(g, p).wait()
                for k_ in range(_GRP):
                    remap(p, k_)
                for k_ in range(_GRP):
                    chunk = chunk_base0 + g * _GRP + k_

                    @pl.when((i > 0) | (p > 0) | (k_ >= _NBUF))
                    def _():
                        for j in range(_IDX_ROWS):
                            out_half(k_, chunk, j).wait()

                    for j in range(_IDX_ROWS):
                        gather_copy(p, k_, j).start()
                    for j in range(_IDX_ROWS):
                        gather_copy(p, k_, j).wait()
                        out_half(k_, chunk, j).start()

                @pl.when(g + 2 < n_grps)
                def _():
                    day_copy(g + 2, p).start()
            return ()

        lax.fori_loop(0, n_iters, body, (), unroll=False)

        for k_ in (_GRP - _NBUF, _GRP - _NBUF + 1):
            chunk = chunk_base0 + (n_grps - 1) * _GRP + k_
            for j in range(_IDX_ROWS):
                out_half(k_, chunk, j).wait()

    return k(day2d, table)


def kernel(day, weekly_pos_embed):
    n, m = day.shape
    total = n * m
    b_per_w = total // _NW
    day2d = day.astype(jnp.int32).reshape(total // 128, 128)
    out = _sc_lookup(day2d, weekly_pos_embed, b_per_w=b_per_w)
    return out.reshape(n, m, 128)
